# baseline (speedup 1.0000x reference)
import functools

import jax
import jax.numpy as jnp
from jax.experimental import pallas as pl
from jax.experimental.pallas import tpu as pltpu
from jax.experimental.pallas import tpu_sc as plsc

N_H = 8
N = 64
NUM_CLS = 117
NUM_OBJ = 80
HUMAN_IDX = 49
DIM = 300
ROW = NUM_CLS * DIM
NUM_KEEP = N_H * N - N_H
PAIRS_PER_STEP = 24
NUM_STEPS = NUM_KEEP // PAIRS_PER_STEP
NUM_WORKERS = 32
PAIRS_PER_WORKER = 16


def _l2n(x):
    return x / jnp.maximum(
        jnp.sqrt(jnp.sum(x * x, axis=1, keepdims=True)), 1e-12)


def _tiles_kernel(ent_ref, rel_ref, nv_ref, hp_o, rn_o, wn_o):
    h_n = _l2n(ent_ref[HUMAN_IDX:HUMAN_IDX + 1, :])
    r_n = _l2n(rel_ref[:])
    w_n = _l2n(nv_ref[:])
    h_dot = jnp.sum(h_n * w_n, axis=1, keepdims=True)
    hp_o[:] = h_n - h_dot * w_n
    rn_o[:] = r_n
    wn_o[:] = w_n


def _sc_replicate(hp_hbm, rn_hbm, wn_hbm, out_h, out_r, out_w,
                  hbuf, rbuf, wbuf, sem):
    c = jax.lax.axis_index("c")
    s = jax.lax.axis_index("s")
    wid = s * 2 + c
    pltpu.sync_copy(hp_hbm, hbuf)
    pltpu.sync_copy(rn_hbm, rbuf)
    pltpu.sync_copy(wn_hbm, wbuf)
    base = wid * 15 + jnp.minimum(wid, 24)

    def _wait3():
        pltpu.make_async_copy(hbuf, out_h.at[base], sem).wait()
        pltpu.make_async_copy(rbuf, out_r.at[base], sem).wait()
        pltpu.make_async_copy(wbuf, out_w.at[base], sem).wait()

    for i in range(PAIRS_PER_WORKER):
        p = jnp.minimum(base + i, NUM_KEEP - 1)
        pltpu.async_copy(hbuf, out_h.at[p], sem)
        pltpu.async_copy(rbuf, out_r.at[p], sem)
        pltpu.async_copy(wbuf, out_w.at[p], sem)
        if i > 0:
            _wait3()
    _wait3()


def _t_stream_kernel(labels_ref, ent_ref, rel_ref, nv_ref,
                     out_t, out_s, tn_s, wn_s, sc_s):
    step = pl.program_id(0)

    @pl.when(step == 0)
    def _compute_core():
        row = jax.lax.broadcasted_iota(jnp.int32, (N, 1), 0)
        labels = jnp.where(row < N_H, HUMAN_IDX, labels_ref[:])
        onehot = (labels == jax.lax.broadcasted_iota(
            jnp.int32, (N, NUM_OBJ), 1)).astype(jnp.float32)
        t_n = _l2n(jnp.dot(onehot, ent_ref[:],
                           preferred_element_type=jnp.float32))
        h_n = _l2n(ent_ref[HUMAN_IDX:HUMAN_IDX + 1, :])
        r_n = _l2n(rel_ref[:])
        w_n = _l2n(nv_ref[:])
        h_dot = jnp.sum(h_n * w_n, axis=1, keepdims=True)
        h_p = h_n - h_dot * w_n
        tn_s[:] = t_n
        wn_s[:] = w_n
        a = h_p + r_n
        ones = jnp.ones((1, DIM), jnp.float32)
        aa_row = jax.lax.dot_general(
            ones, a * a, (((1,), (1,)), ((), ())),
            preferred_element_type=jnp.float32)
        aw_row = jax.lax.dot_general(
            ones, a * w_n, (((1,), (1,)), ((), ())),
            preferred_element_type=jnp.float32)
        t_dot = jax.lax.dot_general(
            t_n, w_n, (((1,), (1,)), ((), ())),
            preferred_element_type=jnp.float32)
        a_t = jax.lax.dot_general(
            t_n, a, (((1,), (1,)), ((), ())),
            preferred_element_type=jnp.float32)
        tt = jnp.sum(t_n * t_n, axis=1, keepdims=True)
        sq = aa_row + tt - t_dot * t_dot - 2.0 * a_t + 2.0 * t_dot * aw_row
        sc_s[:] = jnp.sqrt(jnp.maximum(sq, 0.0))

    wn = wn_s[:]
    i_col = jax.lax.broadcasted_iota(jnp.int32, (PAIRS_PER_STEP, 1), 0)
    k = step * PAIRS_PER_STEP + i_col
    x = k // (N - 1)
    j = k - x * (N - 1)
    y = jnp.where(j >= x, j + 1, j)
    sel = (y == jax.lax.broadcasted_iota(
        jnp.int32, (PAIRS_PER_STEP, N), 1)).astype(jnp.float32)

    out_s[:] = jnp.dot(sel, sc_s[:], preferred_element_type=jnp.float32)
    tn_step = jnp.dot(sel, tn_s[:], preferred_element_type=jnp.float32)

    for i in range(PAIRS_PER_STEP):
        tn = tn_step[i:i + 1, :]
        td = jnp.sum(tn * wn, axis=1, keepdims=True)
        out_t[i] = tn - td * wn


@jax.jit
def kernel(box_labels, ent_emb, rel_emb, norm_vec):
    tile = jax.ShapeDtypeStruct((NUM_CLS, DIM), jnp.float32)
    hp, rn, wn = pl.pallas_call(
        _tiles_kernel,
        out_shape=[tile, tile, tile],
    )(ent_emb, rel_emb, norm_vec)

    mesh = plsc.VectorSubcoreMesh(
        core_axis_name="c", subcore_axis_name="s",
        num_cores=2, num_subcores=16)
    big3 = jax.ShapeDtypeStruct((NUM_KEEP, NUM_CLS, DIM), jnp.float32)
    sc_repl = pl.kernel(
        _sc_replicate,
        out_type=[big3, big3, big3],
        mesh=mesh,
        compiler_params=pltpu.CompilerParams(use_tc_tiling_on_sc=False),
        scratch_types=[
            pltpu.VMEM((NUM_CLS, DIM), jnp.float32),
            pltpu.VMEM((NUM_CLS, DIM), jnp.float32),
            pltpu.VMEM((NUM_CLS, DIM), jnp.float32),
            pltpu.SemaphoreType.DMA,
        ],
    )
    h_keep, r_keep, w_keep = sc_repl(hp, rn, wn)

    labels2d = box_labels.reshape(N, 1)
    t_keep, scores_keep = pl.pallas_call(
        _t_stream_kernel,
        grid=(NUM_STEPS,),
        in_specs=[
            pl.BlockSpec((N, 1), lambda st: (0, 0)),
            pl.BlockSpec((NUM_OBJ, DIM), lambda st: (0, 0)),
            pl.BlockSpec((NUM_CLS, DIM), lambda st: (0, 0)),
            pl.BlockSpec((NUM_CLS, DIM), lambda st: (0, 0)),
        ],
        out_specs=[
            pl.BlockSpec((PAIRS_PER_STEP, NUM_CLS, DIM), lambda st: (st, 0, 0)),
            pl.BlockSpec((PAIRS_PER_STEP, NUM_CLS), lambda st: (st, 0)),
        ],
        out_shape=[
            jax.ShapeDtypeStruct((NUM_KEEP, NUM_CLS, DIM), jnp.float32),
            jax.ShapeDtypeStruct((NUM_KEEP, NUM_CLS), jnp.float32),
        ],
        scratch_shapes=[
            pltpu.VMEM((N, DIM), jnp.float32),
            pltpu.VMEM((NUM_CLS, DIM), jnp.float32),
            pltpu.VMEM((N, NUM_CLS), jnp.float32),
        ],
    )(labels2d, ent_emb, rel_emb, norm_vec)

    return (h_keep, r_keep, w_keep, t_keep, scores_keep)


# speedup vs baseline: 2.9891x; 2.9891x over previous
import functools

import jax
import jax.numpy as jnp
from jax.experimental import pallas as pl
from jax.experimental.pallas import tpu as pltpu
from jax.experimental.pallas import tpu_sc as plsc

N_H = 8
N = 64
NUM_CLS = 117
NUM_OBJ = 80
HUMAN_IDX = 49
DIM = 300
ROW = NUM_CLS * DIM
NUM_KEEP = N_H * N - N_H
PAIRS_PER_STEP = 24
NUM_STEPS = NUM_KEEP // PAIRS_PER_STEP
NUM_WORKERS = 32
PAIRS_PER_WORKER = 16


def _l2n(x):
    return x / jnp.maximum(
        jnp.sqrt(jnp.sum(x * x, axis=1, keepdims=True)), 1e-12)


def _tiles_kernel(ent_ref, rel_ref, nv_ref, hp_o, rn_o, wn_o):
    h_n = _l2n(ent_ref[HUMAN_IDX:HUMAN_IDX + 1, :])
    r_n = _l2n(rel_ref[:])
    w_n = _l2n(nv_ref[:])
    h_dot = jnp.sum(h_n * w_n, axis=1, keepdims=True)
    hp_o[:] = h_n - h_dot * w_n
    rn_o[:] = r_n
    wn_o[:] = w_n


_SC_LAG = 4


def _sc_replicate(hp_hbm, rn_hbm, wn_hbm, out_h, out_r, out_w, buf, sem):
    c = jax.lax.axis_index("c")
    s = jax.lax.axis_index("s")
    wid = s * 2 + c
    oid = jax.lax.rem(wid, 3)
    gidx = jax.lax.div(wid, 3)

    def _do(src, out, n_workers, maxn):
        lo = (NUM_KEEP * gidx) // n_workers
        hi = (NUM_KEEP * (gidx + 1)) // n_workers
        pltpu.sync_copy(src, buf)
        for i in range(maxn):
            p = jnp.minimum(lo + i, hi - 1)
            pltpu.async_copy(buf, out.at[p], sem)
            if i >= _SC_LAG:
                pltpu.make_async_copy(buf, out.at[lo], sem).wait()
        for _ in range(min(_SC_LAG, maxn)):
            pltpu.make_async_copy(buf, out.at[lo], sem).wait()

    @pl.when(oid == 0)
    def _():
        _do(hp_hbm, out_h, 11, 46)

    @pl.when(oid == 1)
    def _():
        _do(rn_hbm, out_r, 11, 46)

    @pl.when(oid == 2)
    def _():
        _do(wn_hbm, out_w, 10, 51)


def _t_stream_kernel(labels_ref, ent_ref, rel_ref, nv_ref,
                     out_t, out_s, tn_s, wn_s, sc_s):
    step = pl.program_id(0)

    @pl.when(step == 0)
    def _compute_core():
        row = jax.lax.broadcasted_iota(jnp.int32, (N, 1), 0)
        labels = jnp.where(row < N_H, HUMAN_IDX, labels_ref[:])
        onehot = (labels == jax.lax.broadcasted_iota(
            jnp.int32, (N, NUM_OBJ), 1)).astype(jnp.float32)
        t_n = _l2n(jnp.dot(onehot, ent_ref[:],
                           preferred_element_type=jnp.float32))
        h_n = _l2n(ent_ref[HUMAN_IDX:HUMAN_IDX + 1, :])
        r_n = _l2n(rel_ref[:])
        w_n = _l2n(nv_ref[:])
        h_dot = jnp.sum(h_n * w_n, axis=1, keepdims=True)
        h_p = h_n - h_dot * w_n
        tn_s[:] = t_n
        wn_s[:] = w_n
        a = h_p + r_n
        ones = jnp.ones((1, DIM), jnp.float32)
        aa_row = jax.lax.dot_general(
            ones, a * a, (((1,), (1,)), ((), ())),
            preferred_element_type=jnp.float32)
        aw_row = jax.lax.dot_general(
            ones, a * w_n, (((1,), (1,)), ((), ())),
            preferred_element_type=jnp.float32)
        t_dot = jax.lax.dot_general(
            t_n, w_n, (((1,), (1,)), ((), ())),
            preferred_element_type=jnp.float32)
        a_t = jax.lax.dot_general(
            t_n, a, (((1,), (1,)), ((), ())),
            preferred_element_type=jnp.float32)
        tt = jnp.sum(t_n * t_n, axis=1, keepdims=True)
        sq = aa_row + tt - t_dot * t_dot - 2.0 * a_t + 2.0 * t_dot * aw_row
        sc_s[:] = jnp.sqrt(jnp.maximum(sq, 0.0))

    wn = wn_s[:]
    i_col = jax.lax.broadcasted_iota(jnp.int32, (PAIRS_PER_STEP, 1), 0)
    k = step * PAIRS_PER_STEP + i_col
    x = k // (N - 1)
    j = k - x * (N - 1)
    y = jnp.where(j >= x, j + 1, j)
    sel = (y == jax.lax.broadcasted_iota(
        jnp.int32, (PAIRS_PER_STEP, N), 1)).astype(jnp.float32)

    out_s[:] = jnp.dot(sel, sc_s[:], preferred_element_type=jnp.float32)
    tn_step = jnp.dot(sel, tn_s[:], preferred_element_type=jnp.float32)

    for i in range(PAIRS_PER_STEP):
        tn = tn_step[i:i + 1, :]
        td = jnp.sum(tn * wn, axis=1, keepdims=True)
        out_t[i] = tn - td * wn


@jax.jit
def kernel(box_labels, ent_emb, rel_emb, norm_vec):
    tile = jax.ShapeDtypeStruct((NUM_CLS, DIM), jnp.float32)
    hp, rn, wn = pl.pallas_call(
        _tiles_kernel,
        out_shape=[tile, tile, tile],
    )(ent_emb, rel_emb, norm_vec)

    mesh = plsc.VectorSubcoreMesh(
        core_axis_name="c", subcore_axis_name="s",
        num_cores=2, num_subcores=16)
    big3 = jax.ShapeDtypeStruct((NUM_KEEP, NUM_CLS, DIM), jnp.float32)
    sc_repl = pl.kernel(
        _sc_replicate,
        out_type=[big3, big3, big3],
        mesh=mesh,
        scratch_types=[
            pltpu.VMEM((NUM_CLS, DIM), jnp.float32),
            pltpu.SemaphoreType.DMA,
        ],
    )
    h_keep, r_keep, w_keep = sc_repl(hp, rn, wn)

    labels2d = box_labels.reshape(N, 1)
    t_keep, scores_keep = pl.pallas_call(
        _t_stream_kernel,
        grid=(NUM_STEPS,),
        in_specs=[
            pl.BlockSpec((N, 1), lambda st: (0, 0)),
            pl.BlockSpec((NUM_OBJ, DIM), lambda st: (0, 0)),
            pl.BlockSpec((NUM_CLS, DIM), lambda st: (0, 0)),
            pl.BlockSpec((NUM_CLS, DIM), lambda st: (0, 0)),
        ],
        out_specs=[
            pl.BlockSpec((PAIRS_PER_STEP, NUM_CLS, DIM), lambda st: (st, 0, 0)),
            pl.BlockSpec((PAIRS_PER_STEP, NUM_CLS), lambda st: (st, 0)),
        ],
        out_shape=[
            jax.ShapeDtypeStruct((NUM_KEEP, NUM_CLS, DIM), jnp.float32),
            jax.ShapeDtypeStruct((NUM_KEEP, NUM_CLS), jnp.float32),
        ],
        scratch_shapes=[
            pltpu.VMEM((N, DIM), jnp.float32),
            pltpu.VMEM((NUM_CLS, DIM), jnp.float32),
            pltpu.VMEM((N, NUM_CLS), jnp.float32),
        ],
    )(labels2d, ent_emb, rel_emb, norm_vec)

    return (h_keep, r_keep, w_keep, t_keep, scores_keep)


# speedup vs baseline: 11.8818x; 3.9751x over previous
import jax
import jax.numpy as jnp
from jax.experimental import pallas as pl
from jax.experimental.pallas import tpu as pltpu

N_H = 8
N = 64
NUM_CLS = 117
NUM_OBJ = 80
HUMAN_IDX = 49
DIM = 300
NUM_KEEP = N_H * N - N_H
C_BLK = 9
C_STEPS = NUM_CLS // C_BLK


def _l2n(x):
    return x / jnp.maximum(
        jnp.sqrt(jnp.sum(x * x, axis=1, keepdims=True)), 1e-12)


def _dot(a, b, dims):
    return jax.lax.dot_general(a, b, (dims, ((), ())),
                               preferred_element_type=jnp.float32)


def _core_kernel(labels_ref, ent_ref, rel_ref, nv_ref,
                 hp_o, rn_o, wn_o, tnp_o, tdp_o, scp_o):
    row = jax.lax.broadcasted_iota(jnp.int32, (N, 1), 0)
    labels = jnp.where(row < N_H, HUMAN_IDX, labels_ref[:])
    onehot = (labels == jax.lax.broadcasted_iota(
        jnp.int32, (N, NUM_OBJ), 1)).astype(jnp.float32)
    t_n = _l2n(_dot(onehot, ent_ref[:], ((1,), (0,))))
    h_n = _l2n(ent_ref[HUMAN_IDX:HUMAN_IDX + 1, :])
    r_n = _l2n(rel_ref[:])
    w_n = _l2n(nv_ref[:])
    h_dot = jnp.sum(h_n * w_n, axis=1, keepdims=True)
    h_p = h_n - h_dot * w_n
    hp_o[:] = h_p
    rn_o[:] = r_n
    wn_o[:] = w_n

    k_row = jax.lax.broadcasted_iota(jnp.int32, (1, NUM_KEEP), 1)
    x = k_row // (N - 1)
    j = k_row - x * (N - 1)
    y_row = jnp.where(j >= x, j + 1, j)
    selT = (y_row == jax.lax.broadcasted_iota(
        jnp.int32, (N, NUM_KEEP), 0)).astype(jnp.float32)

    tnp_o[:] = _dot(t_n, selT, ((0,), (0,)))
    t_dot = _dot(t_n, w_n, ((1,), (1,)))
    tdp_o[:] = _dot(t_dot, selT, ((0,), (0,)))

    a = h_p + r_n
    ones = jnp.ones((1, DIM), jnp.float32)
    aa_row = _dot(ones, a * a, ((1,), (1,)))
    aw_row = _dot(ones, a * w_n, ((1,), (1,)))
    a_t = _dot(t_n, a, ((1,), (1,)))
    tt = jnp.sum(t_n * t_n, axis=1, keepdims=True)
    sq = aa_row + tt - t_dot * t_dot - 2.0 * a_t + 2.0 * t_dot * aw_row
    sc = jnp.sqrt(jnp.maximum(sq, 0.0))
    scp_o[:] = _dot(sc, selT, ((0,), (0,)))


def _bcast_kernel(tile_ref, out_ref):
    out_ref[:] = jnp.broadcast_to(
        tile_ref[0][:, :, None], (C_BLK, DIM, NUM_KEEP))


def _t_kernel(tnp_ref, tdp_ref, wn_ref, out_ref):
    out_ref[:] = (tnp_ref[:][None, :, :]
                  - wn_ref[0][:, :, None] * tdp_ref[0][:, None, :])


def _stream(body, ins, in_specs):
    return pl.pallas_call(
        body,
        grid=(C_STEPS,),
        in_specs=in_specs,
        out_specs=pl.BlockSpec((C_BLK, DIM, NUM_KEEP), lambda c: (c, 0, 0)),
        out_shape=jax.ShapeDtypeStruct((NUM_CLS, DIM, NUM_KEEP), jnp.float32),
    )(*ins)


@jax.jit
def kernel(box_labels, ent_emb, rel_emb, norm_vec):
    labels2d = box_labels.reshape(N, 1)
    hp, rn, wn, tnp, tdp, scp = pl.pallas_call(
        _core_kernel,
        out_shape=[
            jax.ShapeDtypeStruct((NUM_CLS, DIM), jnp.float32),
            jax.ShapeDtypeStruct((NUM_CLS, DIM), jnp.float32),
            jax.ShapeDtypeStruct((NUM_CLS, DIM), jnp.float32),
            jax.ShapeDtypeStruct((DIM, NUM_KEEP), jnp.float32),
            jax.ShapeDtypeStruct((NUM_CLS, NUM_KEEP), jnp.float32),
            jax.ShapeDtypeStruct((NUM_CLS, NUM_KEEP), jnp.float32),
        ],
    )(labels2d, ent_emb, rel_emb, norm_vec)

    hp3 = hp.reshape(C_STEPS, C_BLK, DIM)
    rn3 = rn.reshape(C_STEPS, C_BLK, DIM)
    wn3 = wn.reshape(C_STEPS, C_BLK, DIM)
    tdp3 = tdp.reshape(C_STEPS, C_BLK, NUM_KEEP)
    tile_spec = pl.BlockSpec((1, C_BLK, DIM), lambda c: (c, 0, 0))
    h_t = _stream(_bcast_kernel, (hp3,), [tile_spec])
    r_t = _stream(_bcast_kernel, (rn3,), [tile_spec])
    w_t = _stream(_bcast_kernel, (wn3,), [tile_spec])
    t_t = _stream(
        _t_kernel, (tnp, tdp3, wn3),
        [pl.BlockSpec((DIM, NUM_KEEP), lambda c: (0, 0)),
         pl.BlockSpec((1, C_BLK, NUM_KEEP), lambda c: (c, 0, 0)),
         tile_spec])

    perm = (2, 0, 1)
    return (jnp.transpose(h_t, perm), jnp.transpose(r_t, perm),
            jnp.transpose(w_t, perm), jnp.transpose(t_t, perm),
            jnp.transpose(scp, (1, 0)))


# speedup vs baseline: 12.6723x; 1.0665x over previous
import jax
import jax.numpy as jnp
from jax.experimental import pallas as pl
from jax.experimental.pallas import tpu as pltpu

N_H = 8
N = 64
NUM_CLS = 117
NUM_OBJ = 80
HUMAN_IDX = 49
DIM = 300
NUM_KEEP = N_H * N - N_H
C_BLK = 9
C_STEPS = NUM_CLS // C_BLK


def _l2n(x):
    return x / jnp.maximum(
        jnp.sqrt(jnp.sum(x * x, axis=1, keepdims=True)), 1e-12)


def _dot(a, b, dims):
    return jax.lax.dot_general(a, b, (dims, ((), ())),
                               preferred_element_type=jnp.float32)


def _core_kernel(labels_ref, ent_ref, rel_ref, nv_ref,
                 hp_o, rn_o, wn_o, tnp_o, tdp_o, scp_o):
    row = jax.lax.broadcasted_iota(jnp.int32, (N, 1), 0)
    labels = jnp.where(row < N_H, HUMAN_IDX, labels_ref[:])
    onehot = (labels == jax.lax.broadcasted_iota(
        jnp.int32, (N, NUM_OBJ), 1)).astype(jnp.float32)
    t_n = _l2n(_dot(onehot, ent_ref[:], ((1,), (0,))))
    h_n = _l2n(ent_ref[HUMAN_IDX:HUMAN_IDX + 1, :])
    r_n = _l2n(rel_ref[:])
    w_n = _l2n(nv_ref[:])
    h_dot = jnp.sum(h_n * w_n, axis=1, keepdims=True)
    h_p = h_n - h_dot * w_n
    hp_o[:] = h_p.reshape(C_STEPS, C_BLK, DIM)
    rn_o[:] = r_n.reshape(C_STEPS, C_BLK, DIM)
    wn_o[:] = w_n.reshape(C_STEPS, C_BLK, DIM)

    k_row = jax.lax.broadcasted_iota(jnp.int32, (1, NUM_KEEP), 1)
    x = k_row // (N - 1)
    j = k_row - x * (N - 1)
    y_row = jnp.where(j >= x, j + 1, j)
    selT = (y_row == jax.lax.broadcasted_iota(
        jnp.int32, (N, NUM_KEEP), 0)).astype(jnp.float32)

    tnp_o[:] = _dot(t_n, selT, ((0,), (0,)))
    t_dot = _dot(t_n, w_n, ((1,), (1,)))
    tdp_o[:] = _dot(t_dot, selT, ((0,), (0,))).reshape(
        C_STEPS, C_BLK, NUM_KEEP)

    a = h_p + r_n
    ones = jnp.ones((1, DIM), jnp.float32)
    aa_row = _dot(ones, a * a, ((1,), (1,)))
    aw_row = _dot(ones, a * w_n, ((1,), (1,)))
    a_t = _dot(t_n, a, ((1,), (1,)))
    tt = jnp.sum(t_n * t_n, axis=1, keepdims=True)
    sq = aa_row + tt - t_dot * t_dot - 2.0 * a_t + 2.0 * t_dot * aw_row
    sc = jnp.sqrt(jnp.maximum(sq, 0.0))
    scp_o[:] = _dot(sc, selT, ((0,), (0,)))


def _bcast_kernel(tile_ref, out_ref):
    out_ref[:] = jnp.broadcast_to(
        tile_ref[0][:, :, None], (C_BLK, DIM, NUM_KEEP))


def _t_kernel(tnp_ref, tdp_ref, wn_ref, out_ref):
    out_ref[:] = (tnp_ref[:][None, :, :]
                  - wn_ref[0][:, :, None] * tdp_ref[0][:, None, :])


def _stream(body, ins, in_specs):
    return pl.pallas_call(
        body,
        grid=(C_STEPS,),
        in_specs=in_specs,
        out_specs=pl.BlockSpec((C_BLK, DIM, NUM_KEEP), lambda c: (c, 0, 0)),
        out_shape=jax.ShapeDtypeStruct((NUM_CLS, DIM, NUM_KEEP), jnp.float32),
    )(*ins)


@jax.jit
def kernel(box_labels, ent_emb, rel_emb, norm_vec):
    labels2d = box_labels.reshape(N, 1)
    hp3, rn3, wn3, tnp, tdp3, scp = pl.pallas_call(
        _core_kernel,
        out_shape=[
            jax.ShapeDtypeStruct((C_STEPS, C_BLK, DIM), jnp.float32),
            jax.ShapeDtypeStruct((C_STEPS, C_BLK, DIM), jnp.float32),
            jax.ShapeDtypeStruct((C_STEPS, C_BLK, DIM), jnp.float32),
            jax.ShapeDtypeStruct((DIM, NUM_KEEP), jnp.float32),
            jax.ShapeDtypeStruct((C_STEPS, C_BLK, NUM_KEEP), jnp.float32),
            jax.ShapeDtypeStruct((NUM_CLS, NUM_KEEP), jnp.float32),
        ],
    )(labels2d, ent_emb, rel_emb, norm_vec)

    tile_spec = pl.BlockSpec((1, C_BLK, DIM), lambda c: (c, 0, 0))
    h_t = _stream(_bcast_kernel, (hp3,), [tile_spec])
    r_t = _stream(_bcast_kernel, (rn3,), [tile_spec])
    w_t = _stream(_bcast_kernel, (wn3,), [tile_spec])
    t_t = _stream(
        _t_kernel, (tnp, tdp3, wn3),
        [pl.BlockSpec((DIM, NUM_KEEP), lambda c: (0, 0)),
         pl.BlockSpec((1, C_BLK, NUM_KEEP), lambda c: (c, 0, 0)),
         tile_spec])

    perm = (2, 0, 1)
    return (jnp.transpose(h_t, perm), jnp.transpose(r_t, perm),
            jnp.transpose(w_t, perm), jnp.transpose(t_t, perm),
            jnp.transpose(scp, (1, 0)))
